# baseline (speedup 1.0000x reference)
import functools

import jax
import jax.numpy as jnp
from jax import lax
from jax.experimental import pallas as pl
from jax.experimental.pallas import tpu as pltpu
from jax.experimental.pallas import tpu_sc as plsc

D = 4
IB = 128


def _detile_body(ts_ref, td_ref, os_ref, od_ref):
    for j in range(IB):
        os_ref[4 * j:4 * j + 4, :] = ts_ref[:, 128 * j:128 * j + 128].astype(
            jnp.bfloat16)
        od_ref[4 * j:4 * j + 4, :] = td_ref[:, 128 * j:128 * j + 128].astype(
            jnp.bfloat16)


def _detile(tts, ttd, nt):
    grid = (nt + IB - 1) // IB
    out = jax.ShapeDtypeStruct((4 * nt, 128), jnp.bfloat16)
    return pl.pallas_call(
        _detile_body,
        grid=(grid,),
        in_specs=[pl.BlockSpec((4, 128 * IB), lambda i: (0, i)),
                  pl.BlockSpec((4, 128 * IB), lambda i: (0, i))],
        out_specs=[pl.BlockSpec((4 * IB, 128), lambda i: (i, 0)),
                   pl.BlockSpec((4 * IB, 128), lambda i: (i, 0))],
        out_shape=[out, out],
    )(tts, ttd)


def _body(nc, rows_w, src8, dst8, idx_s_hbm, idx_d_hbm, hist_hbm,
          wb_hbm, out_hbm, ids_raw, idd_raw, gid_s, gid_d,
          s_v, d_v, h_v, o_v, wb_v, sem):
    wid = lax.axis_index("s") * nc + lax.axis_index("c")
    base = wid * rows_w
    nch = rows_w // 16

    pltpu.sync_copy(idx_s_hbm.at[pl.ds(base, rows_w)], ids_raw)
    pltpu.sync_copy(idx_d_hbm.at[pl.ds(base, rows_w)], idd_raw)
    pltpu.sync_copy(hist_hbm.at[pl.ds(base, rows_w)], h_v)
    pltpu.sync_copy(wb_hbm, wb_v)

    def gidx(i, _):
        for raw, gid in ((ids_raw, gid_s), (idd_raw, gid_d)):
            r = raw[pl.ds(i * 16, 16)]
            p0 = (lax.shift_right_logical(r, 7) * 32
                  + lax.shift_right_logical(jnp.bitwise_and(r, 127), 4))
            for k in range(D):
                gid[pl.ds(k * rows_w + i * 16, 16)] = p0 + 8 * k
        return _

    lax.fori_loop(0, nch, gidx, None)

    n_st = (D * rows_w) // 128
    descs = []
    for j in range(n_st):
        sl = pl.ds(j * 128, 128)
        descs.append(pltpu.async_copy(src8.at[gid_s.at[sl]], s_v.at[sl], sem))
        descs.append(pltpu.async_copy(dst8.at[gid_d.at[sl]], d_v.at[sl], sem))
    for dsc in descs:
        dsc.wait()

    iota = lax.iota(jnp.int32, 16)
    zero_c = jnp.zeros((16,), jnp.int32)
    one_c = jnp.ones((16,), jnp.int32)

    himask = jnp.full((16,), -65536, jnp.int32)

    def chunk(i, _):
        rs = ids_raw[pl.ds(i * 16, 16)]
        rd = idd_raw[pl.ds(i * 16, 16)]
        s_j = jnp.bitwise_and(lax.shift_right_logical(rs, 1), 7)
        d_j = jnp.bitwise_and(lax.shift_right_logical(rd, 1), 7)
        s_hi = jnp.bitwise_and(rs, 1)
        d_hi = jnp.bitwise_and(rd, 1)
        acc0 = wb_v[2 * D + 1, 0]
        acc1 = wb_v[2 * D + 1, 1]
        for k in range(D):
            ridx = iota + (k * rows_w + i * 16)
            w = plsc.load_gather(s_v, [ridx, s_j])
            bits = jnp.where(s_hi == 1, jnp.bitwise_and(w, himask),
                             lax.shift_left(w, 16))
            sk = plsc.bitcast(bits, jnp.float32)
            acc0 = acc0 + sk * wb_v[k, 0]
            acc1 = acc1 + sk * wb_v[k, 1]
        for k in range(D):
            ridx = iota + (k * rows_w + i * 16)
            w = plsc.load_gather(d_v, [ridx, d_j])
            bits = jnp.where(d_hi == 1, jnp.bitwise_and(w, himask),
                             lax.shift_left(w, 16))
            dk = plsc.bitcast(bits, jnp.float32)
            acc0 = acc0 + dk * wb_v[D + k, 0]
            acc1 = acc1 + dk * wb_v[D + k, 1]
        h = h_v[pl.ds(i * 16, 16)]
        acc0 = acc0 + h * wb_v[2 * D, 0]
        acc1 = acc1 + h * wb_v[2 * D, 1]
        l = iota + i * 16
        pos = (lax.shift_right_logical(l, 7) * 256
               + jnp.bitwise_and(l, 127))
        plsc.store_scatter(o_v, [pos], acc0)
        plsc.store_scatter(o_v, [pos + 128], acc1)
        return _

    lax.fori_loop(0, nch, chunk, None)
    pltpu.sync_copy(o_v, out_hbm.at[pl.ds(base * 2, rows_w * 2)])


def kernel(src_x, dst_x, src_index, dst_index, history_counts, W, b):
    B = src_index.shape[0]
    n = src_x.shape[0]
    nt = (n + 127) // 128
    mesh = plsc.VectorSubcoreMesh(core_axis_name="c", subcore_axis_name="s")
    nw = mesh.num_cores * mesh.num_subcores
    rows_w = B // nw

    s128, d128 = _detile(src_x.T, dst_x.T, nt)
    src8 = lax.bitcast_convert_type(
        s128.reshape(-1, 2), jnp.int32).reshape(nt * 32, 8)
    dst8 = lax.bitcast_convert_type(
        d128.reshape(-1, 2), jnp.int32).reshape(nt * 32, 8)

    idx_s = src_index.astype(jnp.int32)
    idx_d = dst_index.astype(jnp.int32)
    hist = history_counts.reshape(B)
    wb = jnp.broadcast_to(
        jnp.concatenate([W, b.reshape(1, 2)], axis=0).reshape(2 * D + 2, 2, 1),
        (2 * D + 2, 2, 16)).astype(jnp.float32)

    run = pl.kernel(
        functools.partial(_body, mesh.num_cores, rows_w),
        out_type=jax.ShapeDtypeStruct((2 * B,), jnp.float32),
        mesh=mesh,
        compiler_params=pltpu.CompilerParams(
            needs_layout_passes=False, use_tc_tiling_on_sc=False),
        scratch_types=[
            pltpu.VMEM((rows_w,), jnp.int32),
            pltpu.VMEM((rows_w,), jnp.int32),
            pltpu.VMEM((D * rows_w,), jnp.int32),
            pltpu.VMEM((D * rows_w,), jnp.int32),
            pltpu.VMEM((D * rows_w, 8), jnp.int32),
            pltpu.VMEM((D * rows_w, 8), jnp.int32),
            pltpu.VMEM((rows_w,), jnp.float32),
            pltpu.VMEM((rows_w * 2,), jnp.float32),
            pltpu.VMEM((2 * D + 2, 2, 16), jnp.float32),
            pltpu.SemaphoreType.DMA,
        ],
    )
    out1d = run(src8, dst8, idx_s, idx_d, hist, wb)
    return out1d.reshape(B // 128, 2, 128).transpose(0, 2, 1).reshape(B, 2)


# speedup vs baseline: 38.0855x; 38.0855x over previous
import functools

import jax
import jax.numpy as jnp
from jax import lax
from jax.experimental import pallas as pl
from jax.experimental.pallas import tpu as pltpu
from jax.experimental.pallas import tpu_sc as plsc

D = 4
IB = 128


def _detile_body(ts_ref, td_ref, os_ref, od_ref):
    for j in range(IB):
        os_ref[4 * j:4 * j + 4, :] = ts_ref[:, 128 * j:128 * j + 128]
        od_ref[4 * j:4 * j + 4, :] = td_ref[:, 128 * j:128 * j + 128]


def _detile(tts, ttd, nt):
    grid = (nt + IB - 1) // IB
    out = jax.ShapeDtypeStruct((4 * nt, 128), jnp.float32)
    return pl.pallas_call(
        _detile_body,
        grid=(grid,),
        in_specs=[pl.BlockSpec((4, 128 * IB), lambda i: (0, i)),
                  pl.BlockSpec((4, 128 * IB), lambda i: (0, i))],
        out_specs=[pl.BlockSpec((4 * IB, 128), lambda i: (i, 0)),
                   pl.BlockSpec((4 * IB, 128), lambda i: (i, 0))],
        out_shape=[out, out],
    )(tts, ttd)


def _body(nc, rows_w, src8, dst8, idx_s_hbm, idx_d_hbm, hist_hbm,
          wb_hbm, out_hbm, ids_raw, idd_raw, gid_s, gid_d,
          s_v, d_v, h_v, o_v, wb_v, sem):
    wid = lax.axis_index("s") * nc + lax.axis_index("c")
    base = wid * rows_w
    nch = rows_w // 16

    pltpu.sync_copy(idx_s_hbm.at[pl.ds(base, rows_w)], ids_raw)
    pltpu.sync_copy(idx_d_hbm.at[pl.ds(base, rows_w)], idd_raw)
    pltpu.sync_copy(hist_hbm.at[pl.ds(base, rows_w)], h_v)
    pltpu.sync_copy(wb_hbm, wb_v)

    def gidx(i, _):
        for raw, gid in ((ids_raw, gid_s), (idd_raw, gid_d)):
            r = raw[pl.ds(i * 16, 16)]
            p0 = (lax.shift_right_logical(r, 7) * 64
                  + lax.shift_right_logical(jnp.bitwise_and(r, 127), 3))
            for k in range(D):
                gid[pl.ds(k * rows_w + i * 16, 16)] = p0 + 16 * k
        return _

    lax.fori_loop(0, nch, gidx, None)

    n_st = (D * rows_w) // 128
    descs = []
    for j in range(n_st):
        sl = pl.ds(j * 128, 128)
        descs.append(pltpu.async_copy(src8.at[gid_s.at[sl]], s_v.at[sl], sem))
        descs.append(pltpu.async_copy(dst8.at[gid_d.at[sl]], d_v.at[sl], sem))
    for dsc in descs:
        dsc.wait()

    iota = lax.iota(jnp.int32, 16)
    zero_c = jnp.zeros((16,), jnp.int32)
    one_c = jnp.ones((16,), jnp.int32)

    def chunk(i, _):
        s_j = jnp.bitwise_and(ids_raw[pl.ds(i * 16, 16)], 7)
        d_j = jnp.bitwise_and(idd_raw[pl.ds(i * 16, 16)], 7)
        acc0 = wb_v[2 * D + 1, 0]
        acc1 = wb_v[2 * D + 1, 1]
        for k in range(D):
            ridx = iota + (k * rows_w + i * 16)
            sk = plsc.load_gather(s_v, [ridx, s_j])
            acc0 = acc0 + sk * wb_v[k, 0]
            acc1 = acc1 + sk * wb_v[k, 1]
        for k in range(D):
            ridx = iota + (k * rows_w + i * 16)
            dk = plsc.load_gather(d_v, [ridx, d_j])
            acc0 = acc0 + dk * wb_v[D + k, 0]
            acc1 = acc1 + dk * wb_v[D + k, 1]
        h = h_v[pl.ds(i * 16, 16)]
        acc0 = acc0 + h * wb_v[2 * D, 0]
        acc1 = acc1 + h * wb_v[2 * D, 1]
        l = iota + i * 16
        pos = (lax.shift_right_logical(l, 7) * 256
               + jnp.bitwise_and(l, 127))
        plsc.store_scatter(o_v, [pos], acc0)
        plsc.store_scatter(o_v, [pos + 128], acc1)
        return _

    lax.fori_loop(0, nch, chunk, None)
    pltpu.sync_copy(o_v, out_hbm.at[pl.ds(base * 2, rows_w * 2)])


def kernel(src_x, dst_x, src_index, dst_index, history_counts, W, b):
    B = src_index.shape[0]
    n = src_x.shape[0]
    nt = (n + 127) // 128
    mesh = plsc.VectorSubcoreMesh(core_axis_name="c", subcore_axis_name="s")
    nw = mesh.num_cores * mesh.num_subcores
    rows_w = B // nw

    s128, d128 = _detile(src_x.T, dst_x.T, nt)
    src8 = s128.reshape(nt * 64, 8)
    dst8 = d128.reshape(nt * 64, 8)

    idx_s = src_index.astype(jnp.int32)
    idx_d = dst_index.astype(jnp.int32)
    hist = history_counts.reshape(B)
    wb = jnp.broadcast_to(
        jnp.concatenate([W, b.reshape(1, 2)], axis=0).reshape(2 * D + 2, 2, 1),
        (2 * D + 2, 2, 16)).astype(jnp.float32)

    run = pl.kernel(
        functools.partial(_body, mesh.num_cores, rows_w),
        out_type=jax.ShapeDtypeStruct((2 * B,), jnp.float32),
        mesh=mesh,
        compiler_params=pltpu.CompilerParams(
            needs_layout_passes=False, use_tc_tiling_on_sc=False),
        scratch_types=[
            pltpu.VMEM((rows_w,), jnp.int32),
            pltpu.VMEM((rows_w,), jnp.int32),
            pltpu.VMEM((D * rows_w,), jnp.int32),
            pltpu.VMEM((D * rows_w,), jnp.int32),
            pltpu.VMEM((D * rows_w, 8), jnp.float32),
            pltpu.VMEM((D * rows_w, 8), jnp.float32),
            pltpu.VMEM((rows_w,), jnp.float32),
            pltpu.VMEM((rows_w * 2,), jnp.float32),
            pltpu.VMEM((2 * D + 2, 2, 16), jnp.float32),
            pltpu.SemaphoreType.DMA,
        ],
    )
    out1d = run(src8, dst8, idx_s, idx_d, hist, wb)
    return out1d.reshape(B // 128, 2, 128).transpose(0, 2, 1).reshape(B, 2)
